# baseline (speedup 1.0000x reference)
import jax
import jax.numpy as jnp
from jax import lax
from jax.experimental import pallas as pl
from jax.experimental.pallas import tpu as pltpu
from jax.experimental.pallas import tpu_sc as plsc

K_BINS = 50
N = 1_000_000
NCHUNK = 7
CHUNK = 4_608
ROWS = CHUNK // 128
SPAN = CHUNK * NCHUNK
LAST_BASE = N - SPAN

XY_LO, XY_HI = -4.5, 4.5
Z_LO, Z_HI = -11.0, 11.0
XY_INV = K_BINS / (XY_HI - XY_LO)
Z_INV = K_BINS / (Z_HI - Z_LO)
XY_ADD = -XY_LO * XY_INV + (1.0 - 1e-3)
Z_ADD = -Z_LO * Z_INV + (1.0 - 1e-3)


def _bucketize(v, tbl_ref, mulc, addc):
    t = jnp.clip(v * mulc + addc, 0.0, 52.0)
    c2 = t.astype(jnp.int32)
    bhi = plsc.load_gather(tbl_ref, [c2])
    c = jnp.where(bhi < v, c2, c2 - 1)
    return jnp.minimum(c.astype(jnp.uint32), jnp.uint32(49)).astype(jnp.int32)


def _sc_body(x_hbm, y_hbm, z_hbm, tx_hbm, tz_hbm, p0_hbm, p1_hbm, out_hbm,
             x_a, y_a, z_a, idx_a, x_b, y_b, z_b, idx_b,
             rows0_v, rows1_v, tx_v, tz_v, sem):
    wid = lax.axis_index("s") * 2 + lax.axis_index("c")
    base = jnp.minimum(wid * 31_250 // 8 * 8, LAST_BASE)

    pltpu.sync_copy(tx_hbm, tx_v)
    pltpu.sync_copy(tz_hbm, tz_v)

    lanes = lax.iota(jnp.int32, 16)
    sets = [(x_a, y_a, z_a, idx_a), (x_b, y_b, z_b, idx_b)]

    def dma_in(k, s):
        x_v, y_v, z_v, _ = s
        cbase = base + k * CHUNK
        pltpu.sync_copy(x_hbm.at[pl.ds(cbase, CHUNK)], x_v)
        pltpu.sync_copy(y_hbm.at[pl.ds(cbase, CHUNK)], y_v)
        pltpu.sync_copy(z_hbm.at[pl.ds(cbase, CHUNK)], z_v)

    def compute(s):
        x_v, y_v, z_v, idx_v = s

        def row(r, _):
            for l in range(8):
                o = 128 * r + 16 * l
                xi = _bucketize(x_v[pl.ds(o, 16)], tx_v, XY_INV, XY_ADD)
                yi = _bucketize(y_v[pl.ds(o, 16)], tx_v, XY_INV, XY_ADD)
                zi = _bucketize(z_v[pl.ds(o, 16)], tz_v, Z_INV, Z_ADD)
                flat = (xi * 50 + yi) * 50 + zi
                idx_v[r, pl.ds(16 * l, 16)] = flat >> 3
                z_v[pl.ds(o, 16)] = plsc.bitcast(flat & 7, jnp.float32)
            return 0

        lax.fori_loop(0, ROWS, row, 0)

    def finish(k, s):
        x_v, y_v, z_v, _ = s

        def crow(r, _):
            for l in range(8):
                o = 128 * r + 16 * l
                pv = lanes + o
                col = plsc.bitcast(z_v[pl.ds(o, 16)], jnp.int32)
                x_v[pl.ds(o, 16)] = plsc.load_gather(rows0_v, [pv, col])
                y_v[pl.ds(o, 16)] = plsc.load_gather(rows1_v, [pv, col])
            return 0

        lax.fori_loop(0, ROWS, crow, 0)
        cbase = base + k * CHUNK
        pltpu.sync_copy(x_v, out_hbm.at[0, pl.ds(cbase, CHUNK)])
        pltpu.sync_copy(y_v, out_hbm.at[1, pl.ds(cbase, CHUNK)])

    dma_in(0, sets[0])
    compute(sets[0])
    for k in range(NCHUNK):
        s = sets[k % 2]
        idx_v = s[3]
        if k + 1 < NCHUNK:
            dma_in(k + 1, sets[(k + 1) % 2])
        copies = [
            pltpu.async_copy(src.at[idx_v.at[j]],
                             dst.at[pl.ds(128 * j, 128)], sem)
            for j in range(ROWS)
            for src, dst in ((p0_hbm, rows0_v), (p1_hbm, rows1_v))
        ]
        if k + 1 < NCHUNK:
            compute(sets[(k + 1) % 2])
        for c in copies:
            c.wait()
        finish(k, s)


@jax.jit
def kernel(a, neg_gamma, value):
    inf = jnp.float32(jnp.inf)
    tx = jnp.concatenate([jnp.linspace(XY_LO, XY_HI, K_BINS + 1),
                          jnp.full((13,), inf)])
    tz = jnp.concatenate([jnp.linspace(Z_LO, Z_HI, K_BINS + 1),
                          jnp.full((13,), inf)])
    run = pl.kernel(
        _sc_body,
        out_type=jax.ShapeDtypeStruct((2, N), jnp.float32),
        mesh=plsc.VectorSubcoreMesh(core_axis_name="c", subcore_axis_name="s"),
        compiler_params=pltpu.CompilerParams(needs_layout_passes=False,
                                             use_tc_tiling_on_sc=False),
        scratch_types=[
            pltpu.VMEM((CHUNK,), jnp.float32),
            pltpu.VMEM((CHUNK,), jnp.float32),
            pltpu.VMEM((CHUNK,), jnp.float32),
            pltpu.VMEM((ROWS, 128), jnp.int32),
            pltpu.VMEM((CHUNK,), jnp.float32),
            pltpu.VMEM((CHUNK,), jnp.float32),
            pltpu.VMEM((CHUNK,), jnp.float32),
            pltpu.VMEM((ROWS, 128), jnp.int32),
            pltpu.VMEM((CHUNK, 8), jnp.float32),
            pltpu.VMEM((CHUNK, 8), jnp.float32),
            pltpu.VMEM((64,), jnp.float32),
            pltpu.VMEM((64,), jnp.float32),
            pltpu.SemaphoreType.DMA,
        ],
    )
    planes = jnp.moveaxis(value, 3, 0).reshape(2, -1)
    out = run(a[:, 0], a[:, 1], neg_gamma, tx, tz,
              planes[0].reshape(-1, 8), planes[1].reshape(-1, 8))
    return out.T[None]


# speedup vs baseline: 1.2363x; 1.2363x over previous
import jax
import jax.numpy as jnp
from jax import lax
from jax.experimental import pallas as pl
from jax.experimental.pallas import tpu as pltpu
from jax.experimental.pallas import tpu_sc as plsc

K_BINS = 50
N = 1_000_000
NCHUNK = 7
CHUNK = 4_608
ROWS = CHUNK // 128
SPAN = CHUNK * NCHUNK
LAST_BASE = N - SPAN

NLINES = 15_625
LPW = 489
LAST_LBASE = NLINES - LPW

XY_LO, XY_HI = -4.5, 4.5
Z_LO, Z_HI = -11.0, 11.0
XY_INV = K_BINS / (XY_HI - XY_LO)
Z_INV = K_BINS / (Z_HI - Z_LO)
XY_ADD = -XY_LO * XY_INV + (1.0 - 1e-3)
Z_ADD = -Z_LO * Z_INV + (1.0 - 1e-3)


def _bucketize(v, tbl_ref, mulc, addc):
    t = jnp.clip(v * mulc + addc, 0.0, 52.0)
    c2 = t.astype(jnp.int32)
    bhi = plsc.load_gather(tbl_ref, [c2])
    c = jnp.where(bhi < v, c2, c2 - 1)
    return jnp.minimum(c.astype(jnp.uint32), jnp.uint32(49)).astype(jnp.int32)


def _interleave_body(p0_hbm, p1_hbm, out_hbm, p0_v, p1_v, o_v, w_v):
    wid = lax.axis_index("s") * 2 + lax.axis_index("c")
    base = jnp.minimum(wid * LPW, LAST_LBASE)
    lanes = lax.iota(jnp.int32, 16)
    idx0 = (lanes >> 1) + 16 * (lanes & 1)
    idx1 = idx0 + 8

    pltpu.sync_copy(p0_hbm.at[pl.ds(base * 8, LPW * 8)],
                    p0_v.at[pl.ds(0, LPW * 8)])
    pltpu.sync_copy(p1_hbm.at[pl.ds(base * 8, LPW * 8)],
                    p1_v.at[pl.ds(0, LPW * 8)])

    def pair(m, _):
        o = 16 * m
        w_v[pl.ds(0, 16)] = p0_v[pl.ds(o, 16)]
        w_v[pl.ds(16, 16)] = p1_v[pl.ds(o, 16)]
        o_v[pl.ds(2 * o, 16)] = plsc.load_gather(w_v, [idx0])
        o_v[pl.ds(2 * o + 16, 16)] = plsc.load_gather(w_v, [idx1])
        return 0

    lax.fori_loop(0, LPW // 2, pair, 0)
    o = 8 * (LPW - 1)
    w_v[pl.ds(0, 16)] = p0_v[pl.ds(o, 16)]
    w_v[pl.ds(16, 16)] = p1_v[pl.ds(o, 16)]
    o_v[pl.ds(2 * o, 16)] = plsc.load_gather(w_v, [idx0])
    pltpu.sync_copy(o_v, out_hbm.at[pl.ds(base * 16, LPW * 16)])


def _sc_body(x_hbm, y_hbm, z_hbm, tx_hbm, tz_hbm, val_hbm, out_hbm,
             x_a, y_a, z_a, idx_a, x_b, y_b, z_b, idx_b,
             rows_v, tx_v, tz_v, sem):
    wid = lax.axis_index("s") * 2 + lax.axis_index("c")
    base = jnp.minimum(wid * 31_250 // 8 * 8, LAST_BASE)

    pltpu.sync_copy(tx_hbm, tx_v)
    pltpu.sync_copy(tz_hbm, tz_v)

    lanes = lax.iota(jnp.int32, 16)
    sets = [(x_a, y_a, z_a, idx_a), (x_b, y_b, z_b, idx_b)]

    def dma_in(k, s):
        x_v, y_v, z_v, _ = s
        cbase = base + k * CHUNK
        pltpu.sync_copy(x_hbm.at[pl.ds(cbase, CHUNK)], x_v)
        pltpu.sync_copy(y_hbm.at[pl.ds(cbase, CHUNK)], y_v)
        pltpu.sync_copy(z_hbm.at[pl.ds(cbase, CHUNK)], z_v)

    def compute(s):
        x_v, y_v, z_v, idx_v = s

        def row(r, _):
            for l in range(8):
                o = 128 * r + 16 * l
                xi = _bucketize(x_v[pl.ds(o, 16)], tx_v, XY_INV, XY_ADD)
                yi = _bucketize(y_v[pl.ds(o, 16)], tx_v, XY_INV, XY_ADD)
                zi = _bucketize(z_v[pl.ds(o, 16)], tz_v, Z_INV, Z_ADD)
                flat = (xi * 50 + yi) * 50 + zi
                idx_v[r, pl.ds(16 * l, 16)] = flat >> 3
                z_v[pl.ds(o, 16)] = plsc.bitcast((flat & 7) * 2, jnp.float32)
            return 0

        lax.fori_loop(0, ROWS, row, 0)

    def finish(k, s):
        x_v, y_v, z_v, _ = s

        def crow(r, _):
            for l in range(8):
                o = 128 * r + 16 * l
                pv = lanes + o
                col = plsc.bitcast(z_v[pl.ds(o, 16)], jnp.int32)
                x_v[pl.ds(o, 16)] = plsc.load_gather(rows_v, [pv, col])
                y_v[pl.ds(o, 16)] = plsc.load_gather(rows_v, [pv, col + 1])
            return 0

        lax.fori_loop(0, ROWS, crow, 0)
        cbase = base + k * CHUNK
        pltpu.sync_copy(x_v, out_hbm.at[0, pl.ds(cbase, CHUNK)])
        pltpu.sync_copy(y_v, out_hbm.at[1, pl.ds(cbase, CHUNK)])

    dma_in(0, sets[0])
    compute(sets[0])
    for k in range(NCHUNK):
        s = sets[k % 2]
        idx_v = s[3]
        if k + 1 < NCHUNK:
            dma_in(k + 1, sets[(k + 1) % 2])
        for j in range(ROWS):
            pltpu.async_copy(val_hbm.at[idx_v.at[j]],
                             rows_v.at[pl.ds(128 * j, 128)], sem)
        if k + 1 < NCHUNK:
            compute(sets[(k + 1) % 2])
        pltpu.make_async_copy(val_hbm.at[pl.ds(0, CHUNK)], rows_v, sem).wait()
        finish(k, s)


@jax.jit
def kernel(a, neg_gamma, value):
    inf = jnp.float32(jnp.inf)
    tx = jnp.concatenate([jnp.linspace(XY_LO, XY_HI, K_BINS + 1),
                          jnp.full((13,), inf)])
    tz = jnp.concatenate([jnp.linspace(Z_LO, Z_HI, K_BINS + 1),
                          jnp.full((13,), inf)])
    mesh = plsc.VectorSubcoreMesh(core_axis_name="c", subcore_axis_name="s")
    params = pltpu.CompilerParams(needs_layout_passes=False,
                                  use_tc_tiling_on_sc=False)
    interleave = pl.kernel(
        _interleave_body,
        out_type=jax.ShapeDtypeStruct((NLINES * 16,), jnp.float32),
        mesh=mesh,
        compiler_params=params,
        scratch_types=[
            pltpu.VMEM((LPW * 8 + 8,), jnp.float32),
            pltpu.VMEM((LPW * 8 + 8,), jnp.float32),
            pltpu.VMEM((LPW * 16,), jnp.float32),
            pltpu.VMEM((32,), jnp.float32),
        ],
    )
    run = pl.kernel(
        _sc_body,
        out_type=jax.ShapeDtypeStruct((2, N), jnp.float32),
        mesh=mesh,
        compiler_params=params,
        scratch_types=[
            pltpu.VMEM((CHUNK,), jnp.float32),
            pltpu.VMEM((CHUNK,), jnp.float32),
            pltpu.VMEM((CHUNK,), jnp.float32),
            pltpu.VMEM((ROWS, 128), jnp.int32),
            pltpu.VMEM((CHUNK,), jnp.float32),
            pltpu.VMEM((CHUNK,), jnp.float32),
            pltpu.VMEM((CHUNK,), jnp.float32),
            pltpu.VMEM((ROWS, 128), jnp.int32),
            pltpu.VMEM((CHUNK, 16), jnp.float32),
            pltpu.VMEM((64,), jnp.float32),
            pltpu.VMEM((64,), jnp.float32),
            pltpu.SemaphoreType.DMA,
        ],
    )
    planes = jnp.moveaxis(value, 3, 0).reshape(2, -1)
    val16 = interleave(planes[0], planes[1]).reshape(NLINES, 16)
    out = run(a[:, 0], a[:, 1], neg_gamma, tx, tz, val16)
    return out.T[None]


# speedup vs baseline: 1.4501x; 1.1730x over previous
import jax
import jax.numpy as jnp
from jax import lax
from jax.experimental import pallas as pl
from jax.experimental.pallas import tpu as pltpu
from jax.experimental.pallas import tpu_sc as plsc

K_BINS = 50
N = 1_000_000
NCHUNK = 7
CHUNK = 4_608
ROWS = CHUNK // 128
SPAN = CHUNK * NCHUNK
LAST_BASE = N - SPAN

NLINES = 15_625
LPW = 489
LAST_LBASE = NLINES - LPW

XY_LO, XY_HI = -4.5, 4.5
Z_LO, Z_HI = -11.0, 11.0
XY_INV = K_BINS / (XY_HI - XY_LO)
Z_INV = K_BINS / (Z_HI - Z_LO)
XY_ADD = -XY_LO * XY_INV + (1.0 - 1e-3)
Z_ADD = -Z_LO * Z_INV + (1.0 - 1e-3)


def _bucketize(v, tbl_ref, mulc, addc):
    t = jnp.clip(v * mulc + addc, 0.0, 52.0)
    c2 = t.astype(jnp.int32)
    bhi = plsc.load_gather(tbl_ref, [c2])
    c = jnp.where(bhi < v, c2, c2 - 1)
    return jnp.minimum(c.astype(jnp.uint32), jnp.uint32(49)).astype(jnp.int32)


def _interleave_body(p0_hbm, p1_hbm, out_hbm, p0_v, p1_v, o_v, w_v):
    wid = lax.axis_index("s") * 2 + lax.axis_index("c")
    base = jnp.minimum(wid * LPW, LAST_LBASE)
    lanes = lax.iota(jnp.int32, 16)
    idx0 = (lanes >> 1) + 16 * (lanes & 1)
    idx1 = idx0 + 8

    pltpu.sync_copy(p0_hbm.at[pl.ds(base * 8, LPW * 8)],
                    p0_v.at[pl.ds(0, LPW * 8)])
    pltpu.sync_copy(p1_hbm.at[pl.ds(base * 8, LPW * 8)],
                    p1_v.at[pl.ds(0, LPW * 8)])

    def pair(m, _):
        o = 16 * m
        w_v[pl.ds(0, 16)] = p0_v[pl.ds(o, 16)]
        w_v[pl.ds(16, 16)] = p1_v[pl.ds(o, 16)]
        o_v[pl.ds(2 * o, 16)] = plsc.load_gather(w_v, [idx0])
        o_v[pl.ds(2 * o + 16, 16)] = plsc.load_gather(w_v, [idx1])
        return 0

    lax.fori_loop(0, LPW // 2, pair, 0)
    o = 8 * (LPW - 1)
    w_v[pl.ds(0, 16)] = p0_v[pl.ds(o, 16)]
    w_v[pl.ds(16, 16)] = p1_v[pl.ds(o, 16)]
    o_v[pl.ds(2 * o, 16)] = plsc.load_gather(w_v, [idx0])
    pltpu.sync_copy(o_v, out_hbm.at[pl.ds(base * 16, LPW * 16)])


def _sc_body(xy_hbm, z_hbm, tx_hbm, tz_hbm, val_hbm, out_hbm,
             x_a, y_a, z_a, idx_a, x_b, y_b, z_b, idx_b,
             rows_v, tx_v, tz_v, sem):
    wid = lax.axis_index("s") * 2 + lax.axis_index("c")
    base = jnp.minimum(wid * 31_250 // 8 * 8, LAST_BASE)

    pltpu.sync_copy(tx_hbm, tx_v)
    pltpu.sync_copy(tz_hbm, tz_v)

    lanes = lax.iota(jnp.int32, 16)
    sets = [(x_a, y_a, z_a, idx_a), (x_b, y_b, z_b, idx_b)]

    def dma_in(k, s):
        x_v, y_v, z_v, _ = s
        cbase = base + k * CHUNK
        pltpu.sync_copy(xy_hbm.at[0, pl.ds(cbase, CHUNK)], x_v)
        pltpu.sync_copy(xy_hbm.at[1, pl.ds(cbase, CHUNK)], y_v)
        pltpu.sync_copy(z_hbm.at[pl.ds(cbase, CHUNK)], z_v)

    def compute(s):
        x_v, y_v, z_v, idx_v = s

        def row(r, _):
            for l in range(8):
                o = 128 * r + 16 * l
                xi = _bucketize(x_v[pl.ds(o, 16)], tx_v, XY_INV, XY_ADD)
                yi = _bucketize(y_v[pl.ds(o, 16)], tx_v, XY_INV, XY_ADD)
                zi = _bucketize(z_v[pl.ds(o, 16)], tz_v, Z_INV, Z_ADD)
                flat = (xi * 50 + yi) * 50 + zi
                idx_v[r, pl.ds(16 * l, 16)] = flat >> 3
                z_v[pl.ds(o, 16)] = plsc.bitcast((flat & 7) * 2, jnp.float32)
            return 0

        lax.fori_loop(0, ROWS, row, 0)

    def finish(k, s):
        x_v, y_v, z_v, _ = s

        def crow(r, _):
            for l in range(8):
                o = 128 * r + 16 * l
                pv = lanes + o
                col = plsc.bitcast(z_v[pl.ds(o, 16)], jnp.int32)
                x_v[pl.ds(o, 16)] = plsc.load_gather(rows_v, [pv, col])
                y_v[pl.ds(o, 16)] = plsc.load_gather(rows_v, [pv, col + 1])
            return 0

        lax.fori_loop(0, ROWS, crow, 0)
        cbase = base + k * CHUNK
        pltpu.sync_copy(x_v, out_hbm.at[0, pl.ds(cbase, CHUNK)])
        pltpu.sync_copy(y_v, out_hbm.at[1, pl.ds(cbase, CHUNK)])

    dma_in(0, sets[0])
    compute(sets[0])
    for k in range(NCHUNK):
        s = sets[k % 2]
        idx_v = s[3]
        if k + 1 < NCHUNK:
            dma_in(k + 1, sets[(k + 1) % 2])
        for j in range(ROWS):
            pltpu.async_copy(val_hbm.at[idx_v.at[j]],
                             rows_v.at[pl.ds(128 * j, 128)], sem)
        if k + 1 < NCHUNK:
            compute(sets[(k + 1) % 2])
        pltpu.make_async_copy(val_hbm.at[pl.ds(0, CHUNK)], rows_v, sem).wait()
        finish(k, s)


@jax.jit
def kernel(a, neg_gamma, value):
    inf = jnp.float32(jnp.inf)
    tx = jnp.concatenate([jnp.linspace(XY_LO, XY_HI, K_BINS + 1),
                          jnp.full((13,), inf)])
    tz = jnp.concatenate([jnp.linspace(Z_LO, Z_HI, K_BINS + 1),
                          jnp.full((13,), inf)])
    mesh = plsc.VectorSubcoreMesh(core_axis_name="c", subcore_axis_name="s")
    params = pltpu.CompilerParams(needs_layout_passes=False,
                                  use_tc_tiling_on_sc=False)
    interleave = pl.kernel(
        _interleave_body,
        out_type=jax.ShapeDtypeStruct((NLINES * 16,), jnp.float32),
        mesh=mesh,
        compiler_params=params,
        scratch_types=[
            pltpu.VMEM((LPW * 8 + 8,), jnp.float32),
            pltpu.VMEM((LPW * 8 + 8,), jnp.float32),
            pltpu.VMEM((LPW * 16,), jnp.float32),
            pltpu.VMEM((32,), jnp.float32),
        ],
    )
    run = pl.kernel(
        _sc_body,
        out_type=jax.ShapeDtypeStruct((2, N), jnp.float32),
        mesh=mesh,
        compiler_params=params,
        scratch_types=[
            pltpu.VMEM((CHUNK,), jnp.float32),
            pltpu.VMEM((CHUNK,), jnp.float32),
            pltpu.VMEM((CHUNK,), jnp.float32),
            pltpu.VMEM((ROWS, 128), jnp.int32),
            pltpu.VMEM((CHUNK,), jnp.float32),
            pltpu.VMEM((CHUNK,), jnp.float32),
            pltpu.VMEM((CHUNK,), jnp.float32),
            pltpu.VMEM((ROWS, 128), jnp.int32),
            pltpu.VMEM((CHUNK, 16), jnp.float32),
            pltpu.VMEM((64,), jnp.float32),
            pltpu.VMEM((64,), jnp.float32),
            pltpu.SemaphoreType.DMA,
        ],
    )
    planes = jnp.moveaxis(value, 3, 0).reshape(2, -1)
    val16 = interleave(planes[0], planes[1]).reshape(NLINES, 16)
    out = run(a.T, neg_gamma, tx, tz, val16)
    return out.T[None]


# speedup vs baseline: 1.9472x; 1.3428x over previous
import jax
import jax.numpy as jnp
from jax import lax
from jax.experimental import pallas as pl
from jax.experimental.pallas import tpu as pltpu
from jax.experimental.pallas import tpu_sc as plsc

K_BINS = 50
N = 1_000_000
NCHUNK = 7
CHUNK = 4_608
ROWS = CHUNK // 128
SPAN = CHUNK * NCHUNK
LAST_BASE = N - SPAN

NLINES = 15_625
LPW = 489
LAST_LBASE = NLINES - LPW

XY_LO, XY_HI = -4.5, 4.5
Z_LO, Z_HI = -11.0, 11.0
XY_INV = K_BINS / (XY_HI - XY_LO)
Z_INV = K_BINS / (Z_HI - Z_LO)
XY_ADD = -XY_LO * XY_INV + (1.0 - 1e-3)
Z_ADD = -Z_LO * Z_INV + (1.0 - 1e-3)


def _bucketize(v, tbl_ref, mulc, addc):
    t = jnp.clip(v * mulc + addc, 0.0, 52.0)
    c2 = t.astype(jnp.int32)
    bhi = plsc.load_gather(tbl_ref, [c2])
    c = jnp.where(bhi < v, c2, c2 - 1)
    return jnp.minimum(c.astype(jnp.uint32), jnp.uint32(49)).astype(jnp.int32)


def _interleave_body(p0_hbm, p1_hbm, out_hbm, p0_v, p1_v, o_v, w_v):
    wid = lax.axis_index("s") * 2 + lax.axis_index("c")
    base = jnp.minimum(wid * LPW, LAST_LBASE)
    lanes = lax.iota(jnp.int32, 16)
    idx0 = (lanes >> 1) + 16 * (lanes & 1)
    idx1 = idx0 + 8

    pltpu.sync_copy(p0_hbm.at[pl.ds(base * 8, LPW * 8)],
                    p0_v.at[pl.ds(0, LPW * 8)])
    pltpu.sync_copy(p1_hbm.at[pl.ds(base * 8, LPW * 8)],
                    p1_v.at[pl.ds(0, LPW * 8)])

    def pair(m, _):
        o = 16 * m
        w_v[pl.ds(0, 16)] = p0_v[pl.ds(o, 16)]
        w_v[pl.ds(16, 16)] = p1_v[pl.ds(o, 16)]
        o_v[pl.ds(2 * o, 16)] = plsc.load_gather(w_v, [idx0])
        o_v[pl.ds(2 * o + 16, 16)] = plsc.load_gather(w_v, [idx1])
        return 0

    lax.fori_loop(0, LPW // 2, pair, 0)
    o = 8 * (LPW - 1)
    w_v[pl.ds(0, 16)] = p0_v[pl.ds(o, 16)]
    w_v[pl.ds(16, 16)] = p1_v[pl.ds(o, 16)]
    o_v[pl.ds(2 * o, 16)] = plsc.load_gather(w_v, [idx0])
    pltpu.sync_copy(o_v, out_hbm.at[pl.ds(base * 16, LPW * 16)])


def _sc_body(xy_hbm, z_hbm, tx_hbm, tz_hbm, val_hbm, out_hbm,
             x_a, y_a, z_a, idx_a, x_b, y_b, z_b, idx_b,
             rows_v, tx_v, tz_v, tbl_s, sem):
    wid = lax.axis_index("s") * 2 + lax.axis_index("c")
    base = jnp.minimum(wid * 31_250 // 8 * 8, LAST_BASE)

    pltpu.sync_copy(tx_hbm, tx_v)
    pltpu.sync_copy(tz_hbm, tz_v)

    @pl.when(lax.axis_index("s") == 0)
    def _stage():
        pltpu.sync_copy(val_hbm, tbl_s)

    plsc.subcore_barrier()

    lanes = lax.iota(jnp.int32, 16)
    sets = [(x_a, y_a, z_a, idx_a), (x_b, y_b, z_b, idx_b)]

    def dma_in(k, s):
        x_v, y_v, z_v, _ = s
        cbase = base + k * CHUNK
        pltpu.sync_copy(xy_hbm.at[0, pl.ds(cbase, CHUNK)], x_v)
        pltpu.sync_copy(xy_hbm.at[1, pl.ds(cbase, CHUNK)], y_v)
        pltpu.sync_copy(z_hbm.at[pl.ds(cbase, CHUNK)], z_v)

    def compute(s):
        x_v, y_v, z_v, idx_v = s

        def row(r, _):
            for l in range(8):
                o = 128 * r + 16 * l
                xi = _bucketize(x_v[pl.ds(o, 16)], tx_v, XY_INV, XY_ADD)
                yi = _bucketize(y_v[pl.ds(o, 16)], tx_v, XY_INV, XY_ADD)
                zi = _bucketize(z_v[pl.ds(o, 16)], tz_v, Z_INV, Z_ADD)
                flat = (xi * 50 + yi) * 50 + zi
                idx_v[r, pl.ds(16 * l, 16)] = flat >> 3
                z_v[pl.ds(o, 16)] = plsc.bitcast((flat & 7) * 2, jnp.float32)
            return 0

        lax.fori_loop(0, ROWS, row, 0)

    def finish(k, s):
        x_v, y_v, z_v, _ = s

        def crow(r, _):
            for l in range(8):
                o = 128 * r + 16 * l
                pv = lanes + o
                col = plsc.bitcast(z_v[pl.ds(o, 16)], jnp.int32)
                x_v[pl.ds(o, 16)] = plsc.load_gather(rows_v, [pv, col])
                y_v[pl.ds(o, 16)] = plsc.load_gather(rows_v, [pv, col + 1])
            return 0

        lax.fori_loop(0, ROWS, crow, 0)
        cbase = base + k * CHUNK
        pltpu.sync_copy(x_v, out_hbm.at[0, pl.ds(cbase, CHUNK)])
        pltpu.sync_copy(y_v, out_hbm.at[1, pl.ds(cbase, CHUNK)])

    dma_in(0, sets[0])
    compute(sets[0])
    for k in range(NCHUNK):
        s = sets[k % 2]
        idx_v = s[3]
        if k + 1 < NCHUNK:
            dma_in(k + 1, sets[(k + 1) % 2])
        for j in range(ROWS):
            pltpu.async_copy(tbl_s.at[idx_v.at[j]],
                             rows_v.at[pl.ds(128 * j, 128)], sem)
        if k + 1 < NCHUNK:
            compute(sets[(k + 1) % 2])
        pltpu.make_async_copy(val_hbm.at[pl.ds(0, CHUNK)], rows_v, sem).wait()
        finish(k, s)


@jax.jit
def kernel(a, neg_gamma, value):
    inf = jnp.float32(jnp.inf)
    tx = jnp.concatenate([jnp.linspace(XY_LO, XY_HI, K_BINS + 1),
                          jnp.full((13,), inf)])
    tz = jnp.concatenate([jnp.linspace(Z_LO, Z_HI, K_BINS + 1),
                          jnp.full((13,), inf)])
    mesh = plsc.VectorSubcoreMesh(core_axis_name="c", subcore_axis_name="s")
    params = pltpu.CompilerParams(needs_layout_passes=False,
                                  use_tc_tiling_on_sc=False)
    interleave = pl.kernel(
        _interleave_body,
        out_type=jax.ShapeDtypeStruct((NLINES * 16,), jnp.float32),
        mesh=mesh,
        compiler_params=params,
        scratch_types=[
            pltpu.VMEM((LPW * 8 + 8,), jnp.float32),
            pltpu.VMEM((LPW * 8 + 8,), jnp.float32),
            pltpu.VMEM((LPW * 16,), jnp.float32),
            pltpu.VMEM((32,), jnp.float32),
        ],
    )
    run = pl.kernel(
        _sc_body,
        out_type=jax.ShapeDtypeStruct((2, N), jnp.float32),
        mesh=mesh,
        compiler_params=params,
        scratch_types=[
            pltpu.VMEM((CHUNK,), jnp.float32),
            pltpu.VMEM((CHUNK,), jnp.float32),
            pltpu.VMEM((CHUNK,), jnp.float32),
            pltpu.VMEM((ROWS, 128), jnp.int32),
            pltpu.VMEM((CHUNK,), jnp.float32),
            pltpu.VMEM((CHUNK,), jnp.float32),
            pltpu.VMEM((CHUNK,), jnp.float32),
            pltpu.VMEM((ROWS, 128), jnp.int32),
            pltpu.VMEM((CHUNK, 16), jnp.float32),
            pltpu.VMEM((64,), jnp.float32),
            pltpu.VMEM((64,), jnp.float32),
            pltpu.VMEM_SHARED((NLINES, 16), jnp.float32),
            pltpu.SemaphoreType.DMA,
        ],
    )
    planes = jnp.moveaxis(value, 3, 0).reshape(2, -1)
    val16 = interleave(planes[0], planes[1]).reshape(NLINES, 16)
    out = run(a.T, neg_gamma, tx, tz, val16)
    return out.T[None]
